# baseline (speedup 1.0000x reference)
import functools

import jax
import jax.numpy as jnp
from jax import lax
from jax.experimental import pallas as pl
from jax.experimental.pallas import tpu as pltpu
from jax.experimental.pallas import tpu_sc as plsc

B = 16384
E = 32
IN = 2 * E
L = 16
NC = 2
NS = 16
NW = NC * NS
BPW = B // NW
CH = 128
NCH = BPW // CH
G = BPW // L


def _fold_body(w1_ref, b1_ref, w2_ref, b2_ref, w3_ref, b3_ref, w_ref, c_ref):
    w3 = w3_ref[...]
    w23 = jax.lax.dot(w2_ref[...], w3,
                      precision=jax.lax.Precision.HIGHEST)
    w_ref[...] = jax.lax.dot(w1_ref[...], w23,
                             precision=jax.lax.Precision.HIGHEST)
    c_ref[...] = (
        jax.lax.dot(b1_ref[...], w23, precision=jax.lax.Precision.HIGHEST)
        + jax.lax.dot(b2_ref[...], w3, precision=jax.lax.Precision.HIGHEST)
        + b3_ref[...]
    )


_fold = pl.pallas_call(
    _fold_body,
    out_shape=(
        jax.ShapeDtypeStruct((IN, 1), jnp.float32),
        jax.ShapeDtypeStruct((1, 1), jnp.float32),
    ),
)


_mesh = plsc.VectorSubcoreMesh(core_axis_name="c", subcore_axis_name="s",
                               num_cores=NC, num_subcores=NS)


@functools.partial(
    pl.kernel,
    out_type=jax.ShapeDtypeStruct((B,), jnp.float32),
    mesh=_mesh,
    compiler_params=pltpu.CompilerParams(
        needs_layout_passes=False, use_tc_tiling_on_sc=False),
    scratch_types=[
        pltpu.VMEM((NCH, CH), jnp.int32),
        pltpu.VMEM((NCH, CH), jnp.int32),
        pltpu.VMEM((BPW, E), jnp.float32),
        pltpu.VMEM((BPW, E), jnp.float32),
        pltpu.VMEM((IN, L), jnp.float32),
        pltpu.VMEM((L,), jnp.float32),
        pltpu.VMEM((BPW,), jnp.float32),
        pltpu.SemaphoreType.DMA,
    ],
)
def _sc_main(uf_hbm, if_hbm, ut_hbm, it_hbm, wb_hbm, cb_hbm, out_hbm,
             uidx, iidx, urows, irows, wv, cv, outv, sem):
    wid = lax.axis_index("s") * NC + lax.axis_index("c")
    base = wid * BPW

    copies = []
    for k in range(NCH):
        copies.append(pltpu.async_copy(
            uf_hbm.at[pl.ds(base + k * CH, CH)], uidx.at[k], sem))
        copies.append(pltpu.async_copy(
            if_hbm.at[pl.ds(base + k * CH, CH)], iidx.at[k], sem))
    copies.append(pltpu.async_copy(wb_hbm, wv, sem))
    copies.append(pltpu.async_copy(cb_hbm, cv, sem))
    for c in copies:
        c.wait()

    gathers = []
    for k in range(NCH):
        gathers.append(pltpu.async_copy(
            ut_hbm.at[uidx.at[k]], urows.at[pl.ds(k * CH, CH), :], sem))
        gathers.append(pltpu.async_copy(
            it_hbm.at[iidx.at[k]], irows.at[pl.ds(k * CH, CH), :], sem))
    for g_ in gathers:
        g_.wait()

    lane = lax.iota(jnp.int32, L)
    cvec = cv[...]

    def group(g, carry):
        rows = lane + g * L
        acc = cvec
        for d in range(E):
            cold = jnp.full((L,), d, jnp.int32)
            acc = acc + plsc.load_gather(urows, [rows, cold]) * wv[d, :]
            acc = acc + plsc.load_gather(irows, [rows, cold]) * wv[E + d, :]
        outv[pl.ds(g * L, L)] = 1.0 / (1.0 + jnp.exp(-acc))
        return carry

    lax.fori_loop(0, G, group, 0)
    pltpu.sync_copy(outv, out_hbm.at[pl.ds(base, BPW)])


def kernel(user_feature, item_feature, user_table, item_table,
           W1, b1, W2, b2, W3, b3):
    wf, cf = _fold(W1, b1.reshape(1, -1), W2, b2.reshape(1, -1),
                   W3, b3.reshape(1, 1))
    wbig = jnp.broadcast_to(wf, (IN, L))
    c16 = jnp.broadcast_to(cf.reshape(1), (L,))
    out = _sc_main(user_feature, item_feature, user_table, item_table,
                   wbig, c16)
    return out.reshape(B, 1)
